# baseline (speedup 1.0000x reference)
import functools

import jax
import jax.numpy as jnp
from jax import lax
from jax.experimental import pallas as pl
from jax.experimental.pallas import tpu as pltpu
from jax.experimental.pallas import tpu_sc as plsc

_N = 16384
_D = 512
_NW = 32
_ROWS = _N // _NW
_CH = 64
_NCH = _ROWS // _CH


def _sc_body(idx_hbm, x_hbm, w_hbm, out_hbm,
             buf0, buf1, idx_v, row_v,
             sin0, sin1, sout0, sout1, sem_idx, sem_row):
    wid = lax.axis_index("s") * 2 + lax.axis_index("c")
    base = wid * _ROWS
    bufs = (buf0, buf1)
    sins = (sin0, sin1)
    souts = (sout0, sout1)

    @pl.when(wid == 0)
    def _lookup():
        pltpu.sync_copy(idx_hbm, idx_v)
        pltpu.async_copy(w_hbm.at[idx_v], row_v, sem_idx).wait()
        pltpu.sync_copy(row_v.at[pl.ds(0, 1)], out_hbm.at[pl.ds(_N, 1)])

    in_cps = [None] * _NCH
    out_cps = [None] * _NCH

    def start_in(k):
        b = k % 2
        cp = pltpu.make_async_copy(
            x_hbm.at[pl.ds(base + k * _CH, _CH)], bufs[b], sins[b]
        )
        cp.start()
        in_cps[k] = cp

    def start_out(k):
        b = k % 2
        cp = pltpu.make_async_copy(
            bufs[b], out_hbm.at[pl.ds(base + k * _CH, _CH)], souts[b]
        )
        cp.start()
        out_cps[k] = cp

    start_in(0)
    for k in range(_NCH):
        in_cps[k].wait()
        start_out(k)
        if k + 1 < _NCH:
            if k >= 1:
                out_cps[k - 1].wait()
            start_in(k + 1)
    out_cps[_NCH - 2].wait()
    out_cps[_NCH - 1].wait()


def kernel(input_features, part_cls, obj_cls, W):
    flat = jnp.asarray(part_cls, jnp.int32) * 94 + jnp.asarray(obj_cls, jnp.int32)
    idx = jnp.full((8,), flat, dtype=jnp.int32)
    w2d = W.reshape(94 * 94, _D)
    mesh = plsc.VectorSubcoreMesh(core_axis_name="c", subcore_axis_name="s")
    k = functools.partial(
        pl.kernel,
        mesh=mesh,
        out_type=jax.ShapeDtypeStruct((_N + 1, _D), jnp.float32),
        scratch_types=[
            pltpu.VMEM((_CH, _D), jnp.float32),
            pltpu.VMEM((_CH, _D), jnp.float32),
            pltpu.VMEM((8,), jnp.int32),
            pltpu.VMEM((8, _D), jnp.float32),
            pltpu.SemaphoreType.DMA,
            pltpu.SemaphoreType.DMA,
            pltpu.SemaphoreType.DMA,
            pltpu.SemaphoreType.DMA,
            pltpu.SemaphoreType.DMA,
            pltpu.SemaphoreType.DMA,
        ],
    )(_sc_body)
    return k(idx, input_features, w2d)


# speedup vs baseline: 1.0017x; 1.0017x over previous
import functools

import jax
import jax.numpy as jnp
from jax import lax
from jax.experimental import pallas as pl
from jax.experimental.pallas import tpu as pltpu
from jax.experimental.pallas import tpu_sc as plsc

_N = 16384
_D = 512
_HALF = _N // 2
_CH = 1024
_NCH = _HALF // _CH
_NBUF = 3


def _sc_body(idx_hbm, x_hbm, w_hbm, out_hbm,
             buf0, buf1, buf2, idx_v, row_v,
             sin0, sin1, sin2, sout0, sout1, sout2, sem_idx):
    cid = lax.axis_index("c")
    sid = lax.axis_index("s")
    base = cid * _HALF
    bufs = (buf0, buf1, buf2)
    sins = (sin0, sin1, sin2)
    souts = (sout0, sout1, sout2)

    @pl.when(jnp.logical_and(cid == 0, sid == 0))
    def _lookup():
        pltpu.sync_copy(idx_hbm, idx_v)
        pltpu.async_copy(w_hbm.at[idx_v], row_v, sem_idx).wait()
        pltpu.sync_copy(row_v.at[pl.ds(0, 1)], out_hbm.at[pl.ds(_N, 1)])

    @pl.when(sid == 0)
    def _copy():
        in_cps = [None] * _NCH
        out_cps = [None] * _NCH

        def start_in(k):
            b = k % _NBUF
            cp = pltpu.make_async_copy(
                x_hbm.at[pl.ds(base + k * _CH, _CH)], bufs[b], sins[b]
            )
            cp.start()
            in_cps[k] = cp

        def start_out(k):
            b = k % _NBUF
            cp = pltpu.make_async_copy(
                bufs[b], out_hbm.at[pl.ds(base + k * _CH, _CH)], souts[b]
            )
            cp.start()
            out_cps[k] = cp

        for k in range(_NBUF - 1):
            start_in(k)
        for k in range(_NCH):
            in_cps[k].wait()
            start_out(k)
            if k + _NBUF - 1 < _NCH:
                if k >= 1:
                    out_cps[k - 1].wait()
                start_in(k + _NBUF - 1)
        for k in range(max(0, _NCH - _NBUF), _NCH):
            out_cps[k].wait()


def kernel(input_features, part_cls, obj_cls, W):
    flat = jnp.asarray(part_cls, jnp.int32) * 94 + jnp.asarray(obj_cls, jnp.int32)
    idx = jnp.full((8,), flat, dtype=jnp.int32)
    w2d = W.reshape(94 * 94, _D)
    mesh = plsc.VectorSubcoreMesh(core_axis_name="c", subcore_axis_name="s")
    k = functools.partial(
        pl.kernel,
        mesh=mesh,
        out_type=jax.ShapeDtypeStruct((_N + 1, _D), jnp.float32),
        scratch_types=[
            pltpu.VMEM_SHARED((_CH, _D), jnp.float32),
            pltpu.VMEM_SHARED((_CH, _D), jnp.float32),
            pltpu.VMEM_SHARED((_CH, _D), jnp.float32),
            pltpu.VMEM((8,), jnp.int32),
            pltpu.VMEM((8, _D), jnp.float32),
            pltpu.SemaphoreType.DMA,
            pltpu.SemaphoreType.DMA,
            pltpu.SemaphoreType.DMA,
            pltpu.SemaphoreType.DMA,
            pltpu.SemaphoreType.DMA,
            pltpu.SemaphoreType.DMA,
            pltpu.SemaphoreType.DMA,
        ],
    )(_sc_body)
    return k(idx, input_features, w2d)


# speedup vs baseline: 9.9255x; 9.9090x over previous
import jax
import jax.numpy as jnp
from jax.experimental import pallas as pl
from jax.experimental.pallas import tpu as pltpu

_N = 16384
_D = 512
_CH = 2048
_NCH = _N // _CH
_NBUF = 4


def _concat_body(idx_ref, x_hbm, w_hbm, out_hbm, *scratch):
    bufs = scratch[:_NBUF]
    row_buf = scratch[_NBUF]
    sins = scratch[_NBUF + 1:2 * _NBUF + 1]
    souts = scratch[2 * _NBUF + 1:3 * _NBUF + 1]
    sem_row_in, sem_row_out = scratch[3 * _NBUF + 1:]

    p = idx_ref[0]
    o = idx_ref[1]
    row_in = pltpu.make_async_copy(w_hbm.at[p, o], row_buf, sem_row_in)
    row_in.start()

    in_cps = [None] * _NCH
    out_cps = [None] * _NCH

    def start_in(k):
        b = k % _NBUF
        cp = pltpu.make_async_copy(
            x_hbm.at[pl.ds(k * _CH, _CH)], bufs[b], sins[b]
        )
        cp.start()
        in_cps[k] = cp

    def start_out(k):
        b = k % _NBUF
        cp = pltpu.make_async_copy(
            bufs[b], out_hbm.at[pl.ds(k * _CH, _CH)], souts[b]
        )
        cp.start()
        out_cps[k] = cp

    for k in range(_NBUF - 1):
        start_in(k)
    row_in.wait()
    row_out = pltpu.make_async_copy(
        row_buf, out_hbm.at[pl.ds(_N, 1)], sem_row_out
    )
    row_out.start()
    for k in range(_NCH):
        in_cps[k].wait()
        start_out(k)
        if k + _NBUF - 1 < _NCH:
            if k >= 1:
                out_cps[k - 1].wait()
            start_in(k + _NBUF - 1)
    for k in range(max(0, _NCH - _NBUF), _NCH):
        out_cps[k].wait()
    row_out.wait()


def kernel(input_features, part_cls, obj_cls, W):
    idx = jnp.stack(
        [jnp.asarray(part_cls, jnp.int32), jnp.asarray(obj_cls, jnp.int32)]
    )
    scratch_shapes = (
        [pltpu.VMEM((_CH, _D), jnp.float32)] * _NBUF
        + [pltpu.VMEM((1, _D), jnp.float32)]
        + [pltpu.SemaphoreType.DMA] * (2 * _NBUF + 2)
    )
    return pl.pallas_call(
        _concat_body,
        grid=(),
        in_specs=[
            pl.BlockSpec(memory_space=pltpu.SMEM),
            pl.BlockSpec(memory_space=pl.ANY),
            pl.BlockSpec(memory_space=pl.ANY),
        ],
        out_specs=pl.BlockSpec(memory_space=pl.ANY),
        out_shape=jax.ShapeDtypeStruct((_N + 1, _D), jnp.float32),
        scratch_shapes=scratch_shapes,
    )(idx, input_features, W)


# speedup vs baseline: 9.9647x; 1.0039x over previous
import jax
import jax.numpy as jnp
from jax.experimental import pallas as pl
from jax.experimental.pallas import tpu as pltpu

_N = 16384
_D = 512
_CH = 4096
_NCH = _N // _CH
_NBUF = 3


def _concat_body(idx_ref, x_hbm, w_hbm, out_hbm, *scratch):
    bufs = scratch[:_NBUF]
    row_buf = scratch[_NBUF]
    sins = scratch[_NBUF + 1:2 * _NBUF + 1]
    souts = scratch[2 * _NBUF + 1:3 * _NBUF + 1]
    sem_row_in, sem_row_out = scratch[3 * _NBUF + 1:]

    p = idx_ref[0]
    o = idx_ref[1]
    row_in = pltpu.make_async_copy(w_hbm.at[p, o], row_buf, sem_row_in)
    row_in.start()

    in_cps = [None] * _NCH
    out_cps = [None] * _NCH

    def start_in(k):
        b = k % _NBUF
        cp = pltpu.make_async_copy(
            x_hbm.at[pl.ds(k * _CH, _CH)], bufs[b], sins[b]
        )
        cp.start()
        in_cps[k] = cp

    def start_out(k):
        b = k % _NBUF
        cp = pltpu.make_async_copy(
            bufs[b], out_hbm.at[pl.ds(k * _CH, _CH)], souts[b]
        )
        cp.start()
        out_cps[k] = cp

    for k in range(_NBUF - 1):
        start_in(k)
    row_in.wait()
    row_out = pltpu.make_async_copy(
        row_buf, out_hbm.at[pl.ds(_N, 1)], sem_row_out
    )
    row_out.start()
    for k in range(_NCH):
        in_cps[k].wait()
        start_out(k)
        if k + _NBUF - 1 < _NCH:
            if k >= 1:
                out_cps[k - 1].wait()
            start_in(k + _NBUF - 1)
    for k in range(max(0, _NCH - _NBUF), _NCH):
        out_cps[k].wait()
    row_out.wait()


def kernel(input_features, part_cls, obj_cls, W):
    idx = jnp.stack(
        [jnp.asarray(part_cls, jnp.int32), jnp.asarray(obj_cls, jnp.int32)]
    )
    scratch_shapes = (
        [pltpu.VMEM((_CH, _D), jnp.float32)] * _NBUF
        + [pltpu.VMEM((1, _D), jnp.float32)]
        + [pltpu.SemaphoreType.DMA] * (2 * _NBUF + 2)
    )
    return pl.pallas_call(
        _concat_body,
        grid=(),
        in_specs=[
            pl.BlockSpec(memory_space=pltpu.SMEM),
            pl.BlockSpec(memory_space=pl.ANY),
            pl.BlockSpec(memory_space=pl.ANY),
        ],
        out_specs=pl.BlockSpec(memory_space=pl.ANY),
        out_shape=jax.ShapeDtypeStruct((_N + 1, _D), jnp.float32),
        scratch_shapes=scratch_shapes,
    )(idx, input_features, W)
